# baseline (speedup 1.0000x reference)
import functools

import numpy as np

import jax
import jax.numpy as jnp
from jax import lax
from jax.experimental import pallas as pl
from jax.experimental.pallas import tpu as pltpu
from jax.experimental.pallas import tpu_sc as plsc

N = 10000
E = 320000
D = 128

NC = 2
NS = 16
L = 16
NW = NC * NS
EPW = E // NW
CHUNK = 80
NCHUNK = EPW // CHUNK
NBUF = 2
RING = 2 * NBUF
ZR = 80
NZCH = N // ZR


def _mm_body(x_ref, w_ref, o_ref):
    o_ref[...] = jnp.dot(x_ref[...], w_ref[...],
                         preferred_element_type=jnp.float32
                         ).astype(jnp.bfloat16)


def _matmul(x, w):
    mblk = 2000
    return pl.pallas_call(
        _mm_body,
        grid=(N // mblk,),
        in_specs=[pl.BlockSpec((mblk, D), lambda i: (i, 0)),
                  pl.BlockSpec((D, D), lambda i: (0, 0))],
        out_specs=pl.BlockSpec((mblk, D), lambda i: (i, 0)),
        out_shape=jax.ShapeDtypeStruct((N, D), jnp.bfloat16),
    )(x, w)


_PERM = np.empty((D,), dtype=np.int32)
for _v in range(D // 32):
    for _i in range(16):
        _PERM[32 * _v + 2 * _i] = 32 * _v + _i
        _PERM[32 * _v + 2 * _i + 1] = 32 * _v + 16 + _i


def _sc_body(sup_hbm, pki_hbm, pkw_hbm, out_hbm, acc, *bufs):
    gbuf = bufs[0:NBUF]
    sbuf = bufs[NBUF:2 * NBUF]
    ebuf = bufs[2 * NBUF:2 * NBUF + RING]
    wbuf = bufs[2 * NBUF + RING:2 * NBUF + 2 * RING]
    gsem = bufs[2 * NBUF + 2 * RING:3 * NBUF + 2 * RING]
    ssem = bufs[3 * NBUF + 2 * RING:4 * NBUF + 2 * RING]
    esem = bufs[4 * NBUF + 2 * RING:4 * NBUF + 3 * RING]

    cid = lax.axis_index("c")
    sid = lax.axis_index("s")
    wid = sid * NC + cid
    cbase = wid * NCHUNK

    for c in range(NBUF):
        pltpu.sync_copy(pki_hbm.at[cbase + c], ebuf[c])
        pltpu.sync_copy(pkw_hbm.at[cbase + c], wbuf[c])
    for c in range(NBUF):
        pltpu.async_copy(sup_hbm.at[ebuf[c].at[0, 0]], gbuf[c], gsem[c])

    zero16 = jnp.zeros((L,), jnp.float32)
    for r in range(ZR):
        for v in range(D // L):
            sbuf[0][r, pl.ds(v * L, L)] = zero16

    def _zero(j, carry):
        ch = j * NS + sid

        @pl.when(ch < NZCH)
        def _():
            pltpu.async_copy(sbuf[0], acc.at[pl.ds(ch * ZR, ZR)], ssem[0])
        return carry
    lax.fori_loop(0, (NZCH + NS - 1) // NS, _zero, 0)

    def _zerow(j, carry):
        ch = j * NS + sid

        @pl.when(ch < NZCH)
        def _():
            pltpu.make_async_copy(
                sbuf[0], acc.at[pl.ds(ch * ZR, ZR)], ssem[0]).wait()
        return carry
    lax.fori_loop(0, (NZCH + NS - 1) // NS, _zerow, 0)
    plsc.subcore_barrier()

    def _round(r, carry):
        for s4 in range(RING):
            c = r * RING + s4
            s = s4 % NBUF
            nxt = (s4 + NBUF) % RING

            @pl.when(c < NCHUNK)
            def _(s4=s4, s=s, nxt=nxt, c=c):
                pltpu.make_async_copy(
                    sup_hbm.at[ebuf[s4].at[0, 0]], gbuf[s], gsem[s]).wait()

                @pl.when(c >= NBUF)
                def _():
                    pltpu.make_async_copy(
                        sbuf[s], acc.at[ebuf[nxt].at[1, 0]], ssem[s]).wait()

                @pl.when(c + NBUF < NCHUNK)
                def _():
                    pltpu.async_copy(
                        pki_hbm.at[cbase + c + NBUF], ebuf[nxt], esem[nxt])
                    pltpu.async_copy(
                        pkw_hbm.at[cbase + c + NBUF], wbuf[nxt], esem[nxt])

                def _grp(g, carry2):
                    w16 = wbuf[s4][0, pl.ds(g * L, L)]
                    for j in range(L):
                        bidx = jnp.full((L, 1), j, jnp.int32)
                        wvec = lax.gather(
                            w16, bidx,
                            dimension_numbers=lax.GatherDimensionNumbers(
                                offset_dims=(), collapsed_slice_dims=(0,),
                                start_index_map=(0,)),
                            slice_sizes=(1,),
                            mode=lax.GatherScatterMode.PROMISE_IN_BOUNDS)
                        e = g * L + j
                        for v in range(D // 32):
                            xi = gbuf[s][e, pl.ds(L * v, L)]
                            x32 = plsc.bitcast(xi, jnp.bfloat16)
                            lo, hi = plsc.unpack(
                                x32, format=plsc.PackFormat.INTERLEAVED)
                            sbuf[s][e, pl.ds(32 * v, L)] = lo * wvec
                            sbuf[s][e, pl.ds(32 * v + L, L)] = hi * wvec
                    return carry2
                lax.fori_loop(0, CHUNK // L, _grp, 0)

                pltpu.async_copy(
                    sbuf[s], acc.at[ebuf[s4].at[1, 0]], ssem[s], add=True)

                @pl.when(c + NBUF < NCHUNK)
                def _():
                    pltpu.make_async_copy(
                        pki_hbm.at[cbase + c + NBUF], ebuf[nxt],
                        esem[nxt]).wait()
                    pltpu.make_async_copy(
                        pkw_hbm.at[cbase + c + NBUF], wbuf[nxt],
                        esem[nxt]).wait()
                    pltpu.async_copy(
                        sup_hbm.at[ebuf[nxt].at[0, 0]], gbuf[s], gsem[s])
        return carry
    lax.fori_loop(0, (NCHUNK + RING - 1) // RING, _round, 0)

    for c in range(NCHUNK - NBUF, NCHUNK):
        pltpu.make_async_copy(
            sbuf[c % NBUF], acc.at[ebuf[c % RING].at[1, 0]],
            ssem[c % NBUF]).wait()
    plsc.subcore_barrier()

    def _out(j, carry):
        ch = j * NS + sid

        @pl.when(ch < NZCH)
        def _():
            r0 = ch * ZR
            pltpu.async_copy(acc.at[pl.ds(r0, ZR)],
                             out_hbm.at[cid, pl.ds(r0, ZR)], gsem[0])
        return carry
    lax.fori_loop(0, (NZCH + NS - 1) // NS, _out, 0)

    def _outw(j, carry):
        ch = j * NS + sid

        @pl.when(ch < NZCH)
        def _():
            r0 = ch * ZR
            pltpu.make_async_copy(acc.at[pl.ds(r0, ZR)],
                                  out_hbm.at[cid, pl.ds(r0, ZR)],
                                  gsem[0]).wait()
        return carry
    lax.fori_loop(0, (NZCH + NS - 1) // NS, _outw, 0)


def _sc_aggregate(support, pki, pkw):
    mesh = plsc.VectorSubcoreMesh(core_axis_name="c", subcore_axis_name="s")
    scratch = [pltpu.VMEM_SHARED((N, D), jnp.float32)]
    scratch += [pltpu.VMEM((CHUNK, D // 2), jnp.int32) for _ in range(NBUF)]
    scratch += [pltpu.VMEM((CHUNK, D), jnp.float32) for _ in range(NBUF)]
    scratch += [pltpu.VMEM((2, 1, CHUNK), jnp.int32) for _ in range(RING)]
    scratch += [pltpu.VMEM((1, CHUNK), jnp.float32) for _ in range(RING)]
    scratch += [pltpu.SemaphoreType.DMA for _ in range(2 * NBUF + RING)]
    return pl.kernel(
        _sc_body,
        out_type=jax.ShapeDtypeStruct((NC, N, D), jnp.float32),
        mesh=mesh,
        scratch_types=scratch,
        compiler_params=pltpu.CompilerParams(needs_layout_passes=False, use_tc_tiling_on_sc=False),
    )(support, pki, pkw)


def _add_body(p_ref, b_ref, o_ref):
    o_ref[...] = p_ref[0] + p_ref[1] + b_ref[...]


def _final_add(partial, b2d):
    mblk = 2000
    return pl.pallas_call(
        _add_body,
        grid=(N // mblk,),
        in_specs=[pl.BlockSpec((NC, mblk, D), lambda i: (0, i, 0)),
                  pl.BlockSpec((1, D), lambda i: (0, 0))],
        out_specs=pl.BlockSpec((mblk, D), lambda i: (i, 0)),
        out_shape=jax.ShapeDtypeStruct((N, D), jnp.float32),
    )(partial, b2d)


def kernel(inputs, edge_index, edge_weight, W, b):
    support = lax.bitcast_convert_type(
        _matmul(inputs, W[:, _PERM]).reshape(N, D // 2, 2), jnp.int32)
    col2 = edge_index[1].reshape(NW * NCHUNK, CHUNK)
    row2 = edge_index[0].reshape(NW * NCHUNK, CHUNK)
    pki = jnp.stack([col2, row2], axis=1)[:, :, None, :]
    pkw = edge_weight.reshape(NW * NCHUNK, 1, CHUNK)
    partial = _sc_aggregate(support, pki, pkw)
    return _final_add(partial, b.reshape(1, D))


# speedup vs baseline: 1.8050x; 1.8050x over previous
import functools

import jax
import jax.numpy as jnp
from jax import lax
from jax.experimental import pallas as pl
from jax.experimental.pallas import tpu as pltpu
from jax.experimental.pallas import tpu_sc as plsc

N = 10000
E = 320000
D = 128

NC = 2
NS = 16
L = 16
NW = NC * NS
EPW = E // NW
CHUNK = 80
NCHUNK = EPW // CHUNK
NBUF = 2
RING = 2 * NBUF
ZR = 80
NZCH = N // ZR


def _mm_body(x_ref, w_ref, o_ref):
    o_ref[...] = jnp.dot(x_ref[...], w_ref[...],
                         preferred_element_type=jnp.float32)


def _matmul(x, w):
    mblk = 2000
    return pl.pallas_call(
        _mm_body,
        grid=(N // mblk,),
        in_specs=[pl.BlockSpec((mblk, D), lambda i: (i, 0)),
                  pl.BlockSpec((D, D), lambda i: (0, 0))],
        out_specs=pl.BlockSpec((mblk, D), lambda i: (i, 0)),
        out_shape=jax.ShapeDtypeStruct((N, D), jnp.float32),
    )(x, w)


def _sc_body(sup_hbm, pki_hbm, pkw_hbm, out_hbm, acc, *bufs):
    gbuf = bufs[0:NBUF]
    sbuf = bufs[NBUF:2 * NBUF]
    ebuf = bufs[2 * NBUF:2 * NBUF + RING]
    wbuf = bufs[2 * NBUF + RING:2 * NBUF + 2 * RING]
    gsem = bufs[2 * NBUF + 2 * RING:3 * NBUF + 2 * RING]
    ssem = bufs[3 * NBUF + 2 * RING:4 * NBUF + 2 * RING]
    esem = bufs[4 * NBUF + 2 * RING:4 * NBUF + 3 * RING]

    cid = lax.axis_index("c")
    sid = lax.axis_index("s")
    wid = sid * NC + cid
    cbase = wid * NCHUNK

    for c in range(NBUF):
        pltpu.sync_copy(pki_hbm.at[cbase + c], ebuf[c])
        pltpu.sync_copy(pkw_hbm.at[cbase + c], wbuf[c])
    for c in range(NBUF):
        pltpu.async_copy(sup_hbm.at[ebuf[c].at[0, 0]], gbuf[c], gsem[c])

    zero16 = jnp.zeros((L,), jnp.float32)
    for r in range(ZR):
        for v in range(D // L):
            sbuf[0][r, pl.ds(v * L, L)] = zero16

    def _zero(j, carry):
        ch = j * NS + sid

        @pl.when(ch < NZCH)
        def _():
            pltpu.async_copy(sbuf[0], acc.at[pl.ds(ch * ZR, ZR)], ssem[0])
        return carry
    lax.fori_loop(0, (NZCH + NS - 1) // NS, _zero, 0)

    def _zerow(j, carry):
        ch = j * NS + sid

        @pl.when(ch < NZCH)
        def _():
            pltpu.make_async_copy(
                sbuf[0], acc.at[pl.ds(ch * ZR, ZR)], ssem[0]).wait()
        return carry
    lax.fori_loop(0, (NZCH + NS - 1) // NS, _zerow, 0)
    plsc.subcore_barrier()

    def _round(r, carry):
        for s4 in range(RING):
            c = r * RING + s4
            s = s4 % NBUF
            nxt = (s4 + NBUF) % RING

            @pl.when(c < NCHUNK)
            def _(s4=s4, s=s, nxt=nxt, c=c):
                pltpu.make_async_copy(
                    sup_hbm.at[ebuf[s4].at[0, 0]], gbuf[s], gsem[s]).wait()

                @pl.when(c >= NBUF)
                def _():
                    pltpu.make_async_copy(
                        sbuf[s], acc.at[ebuf[nxt].at[1, 0]], ssem[s]).wait()

                @pl.when(c + NBUF < NCHUNK)
                def _():
                    pltpu.async_copy(
                        pki_hbm.at[cbase + c + NBUF], ebuf[nxt], esem[nxt])
                    pltpu.async_copy(
                        pkw_hbm.at[cbase + c + NBUF], wbuf[nxt], esem[nxt])

                def _grp(g, carry2):
                    w16 = wbuf[s4][0, pl.ds(g * L, L)]
                    for j in range(L):
                        bidx = jnp.full((L, 1), j, jnp.int32)
                        wvec = lax.gather(
                            w16, bidx,
                            dimension_numbers=lax.GatherDimensionNumbers(
                                offset_dims=(), collapsed_slice_dims=(0,),
                                start_index_map=(0,)),
                            slice_sizes=(1,),
                            mode=lax.GatherScatterMode.PROMISE_IN_BOUNDS)
                        e = g * L + j
                        for v in range(D // L):
                            sbuf[s][e, pl.ds(v * L, L)] = (
                                gbuf[s][e, pl.ds(v * L, L)] * wvec)
                    return carry2
                lax.fori_loop(0, CHUNK // L, _grp, 0)

                pltpu.async_copy(
                    sbuf[s], acc.at[ebuf[s4].at[1, 0]], ssem[s], add=True)

                @pl.when(c + NBUF < NCHUNK)
                def _():
                    pltpu.make_async_copy(
                        pki_hbm.at[cbase + c + NBUF], ebuf[nxt],
                        esem[nxt]).wait()
                    pltpu.make_async_copy(
                        pkw_hbm.at[cbase + c + NBUF], wbuf[nxt],
                        esem[nxt]).wait()
                    pltpu.async_copy(
                        sup_hbm.at[ebuf[nxt].at[0, 0]], gbuf[s], gsem[s])
        return carry
    lax.fori_loop(0, (NCHUNK + RING - 1) // RING, _round, 0)

    for c in range(NCHUNK - NBUF, NCHUNK):
        pltpu.make_async_copy(
            sbuf[c % NBUF], acc.at[ebuf[c % RING].at[1, 0]],
            ssem[c % NBUF]).wait()
    plsc.subcore_barrier()

    def _out(j, carry):
        ch = j * NS + sid

        @pl.when(ch < NZCH)
        def _():
            r0 = ch * ZR
            pltpu.async_copy(acc.at[pl.ds(r0, ZR)],
                             out_hbm.at[cid, pl.ds(r0, ZR)], gsem[0])
        return carry
    lax.fori_loop(0, (NZCH + NS - 1) // NS, _out, 0)

    def _outw(j, carry):
        ch = j * NS + sid

        @pl.when(ch < NZCH)
        def _():
            r0 = ch * ZR
            pltpu.make_async_copy(acc.at[pl.ds(r0, ZR)],
                                  out_hbm.at[cid, pl.ds(r0, ZR)],
                                  gsem[0]).wait()
        return carry
    lax.fori_loop(0, (NZCH + NS - 1) // NS, _outw, 0)


def _sc_aggregate(support, pki, pkw):
    mesh = plsc.VectorSubcoreMesh(core_axis_name="c", subcore_axis_name="s")
    scratch = [pltpu.VMEM_SHARED((N, D), jnp.float32)]
    scratch += [pltpu.VMEM((CHUNK, D), jnp.float32) for _ in range(NBUF)]
    scratch += [pltpu.VMEM((CHUNK, D), jnp.float32) for _ in range(NBUF)]
    scratch += [pltpu.VMEM((2, 1, CHUNK), jnp.int32) for _ in range(RING)]
    scratch += [pltpu.VMEM((1, CHUNK), jnp.float32) for _ in range(RING)]
    scratch += [pltpu.SemaphoreType.DMA for _ in range(2 * NBUF + RING)]
    return pl.kernel(
        _sc_body,
        out_type=jax.ShapeDtypeStruct((NC, N, D), jnp.float32),
        mesh=mesh,
        scratch_types=scratch,
        compiler_params=pltpu.CompilerParams(needs_layout_passes=False),
    )(support, pki, pkw)


def _add_body(p_ref, b_ref, o_ref):
    o_ref[...] = p_ref[0] + p_ref[1] + b_ref[...]


def _final_add(partial, b2d):
    mblk = 2000
    return pl.pallas_call(
        _add_body,
        grid=(N // mblk,),
        in_specs=[pl.BlockSpec((NC, mblk, D), lambda i: (0, i, 0)),
                  pl.BlockSpec((1, D), lambda i: (0, 0))],
        out_specs=pl.BlockSpec((mblk, D), lambda i: (i, 0)),
        out_shape=jax.ShapeDtypeStruct((N, D), jnp.float32),
    )(partial, b2d)


def kernel(inputs, edge_index, edge_weight, W, b):
    support = _matmul(inputs, W)
    col2 = edge_index[1].reshape(NW * NCHUNK, CHUNK)
    row2 = edge_index[0].reshape(NW * NCHUNK, CHUNK)
    pki = jnp.stack([col2, row2], axis=1)[:, :, None, :]
    pkw = edge_weight.reshape(NW * NCHUNK, 1, CHUNK)
    partial = _sc_aggregate(support, pki, pkw)
    return _final_add(partial, b.reshape(1, D))
